# Initial kernel scaffold; baseline (speedup 1.0000x reference)
import jax
import jax.numpy as jnp
from jax.experimental import pallas as pl


def kernel(patch_verts, faces, rotations, translations, moving_idx, static_idx, handle_value, alternation):
    raise NotImplementedError("write your pallas kernel here")


# speedup vs baseline: 3.5088x; 3.5088x over previous
import jax
import jax.numpy as jnp
from jax.experimental import pallas as pl
from jax.experimental.pallas import tpu as pltpu

_P, _N, _M = 16, 1024, 2048
_E = 3 * _M
_FC = 512
_DC = 512
_EB = 256

_F32 = jnp.float32
_PREC = jax.lax.Precision.HIGHEST


def _fiota(shape, dim):
    return jax.lax.broadcasted_iota(jnp.int32, shape, dim).astype(_F32)


def _dot(a, b):
    return jnp.dot(a, b, preferred_element_type=_F32, precision=_PREC)


def _dot_nt(a, b):
    return jax.lax.dot_general(a, b, (((1,), (1,)), ((), ())),
                               preferred_element_type=_F32, precision=_PREC)


def _to_col(row):
    k = row.shape[1]
    ident = (_fiota((k, 1), 0) == _fiota((1, k), 1)).astype(_F32)
    return _dot_nt(ident, row)


def _body(x2_ref, r2_ref, t2_ref, f_ref, mi_ref, si_ref, hv_ref, out_ref,
          f0_ref, f1_ref, f2_ref, if_ref, jf_ref, cot_ref, d_ref):
    x2 = x2_ref[...]
    r2 = r2_ref[...]
    t2 = t2_ref[...]
    f = f_ref[...]
    hv = hv_ref[...]
    iota_n = _fiota((1, _N), 1)

    tr_blocks = []
    for c in range(3):
        acc = t2[:, c * 16:(c + 1) * 16]
        for m in range(3):
            acc = acc + r2[:, ((3 * c + m) * 16):((3 * c + m + 1) * 16)] * x2[:, m * 16:(m + 1) * 16]
        tr_blocks.append(acc)
    tr = jnp.concatenate(tr_blocks, axis=1)

    f0 = f[0:1, :]
    f1 = f[1:2, :]
    f2 = f[2:3, :]
    i_row = jnp.concatenate([f0, f1, f0], axis=1)
    j_row = jnp.concatenate([f1, f2, f2], axis=1)
    key_row = i_row * float(_N) + j_row
    rkey_row = j_row * float(_N) + i_row
    for r in range(_M // _FC):
        f0_ref[r:r + 1, :] = f0[:, r * _FC:(r + 1) * _FC]
        f1_ref[r:r + 1, :] = f1[:, r * _FC:(r + 1) * _FC]
        f2_ref[r:r + 1, :] = f2[:, r * _FC:(r + 1) * _FC]
    for r in range(_E // _DC):
        if_ref[r:r + 1, :] = i_row[:, r * _DC:(r + 1) * _DC]
        jf_ref[r:r + 1, :] = j_row[:, r * _DC:(r + 1) * _DC]

    def _cot_step(c, carry):
        v = []
        for fr in (f0_ref, f1_ref, f2_ref):
            idx = _to_col(fr[pl.ds(c, 1), :])
            oh = (idx == iota_n).astype(_F32)
            v.append(_dot(oh, x2))

        def _sidelen(a, b):
            s = None
            for cc in range(3):
                dd = a[:, cc * 16:(cc + 1) * 16] - b[:, cc * 16:(cc + 1) * 16]
                s = dd * dd if s is None else s + dd * dd
            return jnp.sqrt(s)

        la = _sidelen(v[1], v[2])
        lb = _sidelen(v[0], v[2])
        lc = _sidelen(v[0], v[1])
        s = 0.5 * (la + lb + lc)
        area = jnp.sqrt(jnp.maximum(s * (s - la) * (s - lb) * (s - lc), 1e-12))
        a2, b2, c2 = la * la, lb * lb, lc * lc
        cot_ref[pl.ds(c * _FC, _FC), :] = (b2 + c2 - a2) / area * 0.25
        cot_ref[pl.ds(_M + c * _FC, _FC), :] = (a2 + c2 - b2) / area * 0.25
        cot_ref[pl.ds(2 * _M + c * _FC, _FC), :] = (a2 + b2 - c2) / area * 0.25
        return carry

    jax.lax.fori_loop(0, _M // _FC, _cot_step, 0)

    def _d_step(c, carry):
        jc = _to_col(jf_ref[pl.ds(c, 1), :])
        ic = _to_col(if_ref[pl.ds(c, 1), :])
        oh_j = (jc == iota_n).astype(_F32)
        oh_i = (ic == iota_n).astype(_F32)
        xj = _dot(oh_j, x2)
        trj = _dot(oh_j, tr)
        ri = _dot(oh_i, r2)
        d = None
        for cc in range(3):
            acc = trj[:, cc * 16:(cc + 1) * 16]
            for m in range(3):
                acc = acc - ri[:, ((3 * cc + m) * 16):((3 * cc + m + 1) * 16)] * xj[:, m * 16:(m + 1) * 16]
            d = acc * acc if d is None else d + acc * acc
        d_ref[pl.ds(c * _DC, _DC), :] = d
        return carry

    jax.lax.fori_loop(0, _E // _DC, _d_step, 0)

    d_full = d_ref[...]
    iota_e = _fiota((1, _E), 1)
    ones_e = jnp.full((1, _E), 1.0, dtype=_F32)

    def _arap_step(b, acc):
        erow = b.astype(_F32) * float(_EB) + _fiota((_EB, 1), 0)
        sel = (iota_e == erow).astype(_F32)
        kb = _dot_nt(sel, key_row)
        rkb = _dot_nt(sel, rkey_row)
        eqf = (kb == key_row)
        eqr = (rkb == key_row)
        later = jnp.logical_and(eqf, iota_e > erow).astype(_F32)
        cnt_after = _dot_nt(later, ones_e)
        mask_last = (cnt_after == 0.0).astype(_F32)
        dfwd = _dot(eqf.astype(_F32), d_full)
        drev = _dot(eqr.astype(_F32), d_full)
        cot_b = cot_ref[pl.ds(b * _EB, _EB), :]
        contrib = jnp.sum(mask_last * cot_b * (dfwd + drev), axis=0)
        return acc + contrib.reshape(1, _P)

    arap_acc = jax.lax.fori_loop(0, _E // _EB, _arap_step,
                                 jnp.zeros((1, _P), dtype=_F32))
    arap = jnp.sum(arap_acc) / float(_P)

    iota_p = _fiota((1, _P), 1)

    def _handle_loss(hidx_ref):
        pcol = _to_col(hidx_ref[0:1, :])
        ncol = _to_col(hidx_ref[1:2, :])
        pm = (pcol == iota_p).astype(_F32)
        oh = (ncol == iota_n).astype(_F32)
        xg = _dot(oh, x2)
        tg = _dot(oh, t2)
        rg = _dot(oh, r2)
        xs = [jnp.sum(pm * xg[:, m * 16:(m + 1) * 16], axis=1) for m in range(3)]
        sq = None
        for c in range(3):
            pos = jnp.sum(pm * tg[:, c * 16:(c + 1) * 16], axis=1)
            for m in range(3):
                r_cm = jnp.sum(pm * rg[:, ((3 * c + m) * 16):((3 * c + m + 1) * 16)], axis=1)
                pos = pos + r_cm * xs[m]
            hv_c = jnp.sum(pm * hv[c, :].reshape(1, _P), axis=1)
            e = pos - hv_c
            sq = e * e if sq is None else sq + e * e
        return jnp.sum(sq) / (64.0 * 3.0)

    moving_loss = _handle_loss(mi_ref)
    static_loss = _handle_loss(si_ref)

    oi = _fiota((1, 128), 1)
    out_ref[...] = (jnp.where(oi == 0.0, arap, 0.0)
                    + jnp.where(oi == 1.0, moving_loss, 0.0)
                    + jnp.where(oi == 2.0, static_loss, 0.0))


def kernel(patch_verts, faces, rotations, translations, moving_idx, static_idx,
           handle_value, alternation):
    del alternation
    x2 = jnp.transpose(patch_verts, (1, 2, 0)).reshape(_N, 3 * _P)
    r2 = jnp.transpose(rotations, (1, 2, 3, 0)).reshape(_N, 9 * _P)
    t2 = jnp.transpose(translations, (1, 2, 0)).reshape(_N, 3 * _P)
    f = faces.T.astype(_F32)
    mi = moving_idx.T.astype(_F32)
    si = static_idx.T.astype(_F32)
    hv = handle_value.T
    out = pl.pallas_call(
        _body,
        out_shape=jax.ShapeDtypeStruct((1, 128), _F32),
        scratch_shapes=[
            pltpu.VMEM((_M // _FC, _FC), _F32),
            pltpu.VMEM((_M // _FC, _FC), _F32),
            pltpu.VMEM((_M // _FC, _FC), _F32),
            pltpu.VMEM((_E // _DC, _DC), _F32),
            pltpu.VMEM((_E // _DC, _DC), _F32),
            pltpu.VMEM((_E, _P), _F32),
            pltpu.VMEM((_E, _P), _F32),
        ],
    )(x2, r2, t2, f, mi, si, hv)
    return out[0, :3]


# speedup vs baseline: 9.2213x; 2.6281x over previous
import jax
import jax.numpy as jnp
from jax.experimental import pallas as pl
from jax.experimental.pallas import tpu as pltpu

_P, _N, _M = 16, 1024, 2048
_E = 3 * _M
_FC = 512
_DC = 512
_EB = 256

_F32 = jnp.float32
_PREC = jax.lax.Precision.HIGHEST


def _fiota(shape, dim):
    return jax.lax.broadcasted_iota(jnp.int32, shape, dim).astype(_F32)


_DEF = jax.lax.Precision.DEFAULT


def _dot(a, b, prec=_PREC):
    return jnp.dot(a, b, preferred_element_type=_F32, precision=prec)


def _dot_nt(a, b, prec=_PREC):
    return jax.lax.dot_general(a, b, (((1,), (1,)), ((), ())),
                               preferred_element_type=_F32, precision=prec)


def _split(v):
    hi = v.astype(jnp.bfloat16).astype(_F32)
    return hi, v - hi


def _dot2(onehot, hi, lo):
    return _dot(onehot, hi, _DEF) + _dot(onehot, lo, _DEF)


def _to_col(row):
    k = row.shape[1]
    ident = (_fiota((k, 1), 0) == _fiota((1, k), 1)).astype(_F32)
    return _dot_nt(ident, row)


def _body(x2_ref, r2_ref, t2_ref, f_ref, mi_ref, si_ref, hv_ref, out_ref,
          f0_ref, f1_ref, f2_ref, if_ref, jf_ref, cot_ref, d_ref):
    x2 = x2_ref[...]
    r2 = r2_ref[...]
    t2 = t2_ref[...]
    f = f_ref[...]
    hv = hv_ref[...]
    iota_n = _fiota((1, _N), 1)

    tr_blocks = []
    for c in range(3):
        acc = t2[:, c * 16:(c + 1) * 16]
        for m in range(3):
            acc = acc + r2[:, ((3 * c + m) * 16):((3 * c + m + 1) * 16)] * x2[:, m * 16:(m + 1) * 16]
        tr_blocks.append(acc)
    tr = jnp.concatenate(tr_blocks, axis=1)

    f0 = f[0:1, :]
    f1 = f[1:2, :]
    f2 = f[2:3, :]
    i_row = jnp.concatenate([f0, f1, f0], axis=1)
    j_row = jnp.concatenate([f1, f2, f2], axis=1)
    key_row = i_row * float(_N) + j_row
    for r in range(_M // _FC):
        f0_ref[r:r + 1, :] = f0[:, r * _FC:(r + 1) * _FC]
        f1_ref[r:r + 1, :] = f1[:, r * _FC:(r + 1) * _FC]
        f2_ref[r:r + 1, :] = f2[:, r * _FC:(r + 1) * _FC]
    for r in range(_E // _DC):
        if_ref[r:r + 1, :] = i_row[:, r * _DC:(r + 1) * _DC]
        jf_ref[r:r + 1, :] = j_row[:, r * _DC:(r + 1) * _DC]

    def _cot_step(c, carry):
        v = []
        for fr in (f0_ref, f1_ref, f2_ref):
            idx = _to_col(fr[pl.ds(c, 1), :])
            oh = (idx == iota_n).astype(_F32)
            v.append(_dot(oh, x2))

        def _sidelen(a, b):
            s = None
            for cc in range(3):
                dd = a[:, cc * 16:(cc + 1) * 16] - b[:, cc * 16:(cc + 1) * 16]
                s = dd * dd if s is None else s + dd * dd
            return jnp.sqrt(s)

        la = _sidelen(v[1], v[2])
        lb = _sidelen(v[0], v[2])
        lc = _sidelen(v[0], v[1])
        s = 0.5 * (la + lb + lc)
        area = jnp.sqrt(jnp.maximum(s * (s - la) * (s - lb) * (s - lc), 1e-12))
        a2, b2, c2 = la * la, lb * lb, lc * lc
        cot_ref[pl.ds(c * _FC, _FC), :] = (b2 + c2 - a2) / area * 0.25
        cot_ref[pl.ds(_M + c * _FC, _FC), :] = (a2 + c2 - b2) / area * 0.25
        cot_ref[pl.ds(2 * _M + c * _FC, _FC), :] = (a2 + b2 - c2) / area * 0.25
        return carry

    jax.lax.fori_loop(0, _M // _FC, _cot_step, 0)

    x2h, x2l = _split(x2)
    trh, trl = _split(tr)
    r2h, r2l = _split(r2)

    def _d_step(c, carry):
        jc = _to_col(jf_ref[pl.ds(c, 1), :])
        ic = _to_col(if_ref[pl.ds(c, 1), :])
        oh_j = (jc == iota_n).astype(_F32)
        oh_i = (ic == iota_n).astype(_F32)
        xj = _dot2(oh_j, x2h, x2l)
        trj = _dot2(oh_j, trh, trl)
        ri = _dot2(oh_i, r2h, r2l)
        d = None
        for cc in range(3):
            acc = trj[:, cc * 16:(cc + 1) * 16]
            for m in range(3):
                acc = acc - ri[:, ((3 * cc + m) * 16):((3 * cc + m + 1) * 16)] * xj[:, m * 16:(m + 1) * 16]
            d = acc * acc if d is None else d + acc * acc
        d_ref[pl.ds(c * _DC, _DC), :] = d
        return carry

    jax.lax.fori_loop(0, _E // _DC, _d_step, 0)

    d_hi, d_lo = _split(d_ref[...])
    iota_e = _fiota((1, _E), 1)
    ones_e = jnp.full((1, _E), 1.0, dtype=_F32)

    def _arap_step(b, acc):
        erow = b.astype(_F32) * float(_EB) + _fiota((_EB, 1), 0)
        sel = (iota_e == erow).astype(_F32)
        ib = _dot_nt(sel, i_row)
        jb = _dot_nt(sel, j_row)
        kb = ib * float(_N) + jb
        rkb = jb * float(_N) + ib
        eqf = (kb == key_row)
        eqr = (rkb == key_row)
        later = jnp.logical_and(eqf, iota_e > erow).astype(_F32)
        cnt_after = _dot_nt(later, ones_e, _DEF)
        mask_last = (cnt_after == 0.0).astype(_F32)
        eqsum = eqf.astype(_F32) + eqr.astype(_F32)
        dsum = _dot2(eqsum, d_hi, d_lo)
        cot_b = cot_ref[pl.ds(b * _EB, _EB), :]
        contrib = jnp.sum(mask_last * cot_b * dsum, axis=0)
        return acc + contrib.reshape(1, _P)

    arap_acc = jax.lax.fori_loop(0, _E // _EB, _arap_step,
                                 jnp.zeros((1, _P), dtype=_F32))
    arap = jnp.sum(arap_acc) / float(_P)

    iota_p = _fiota((1, _P), 1)

    def _handle_loss(hidx_ref):
        pcol = _to_col(hidx_ref[0:1, :])
        ncol = _to_col(hidx_ref[1:2, :])
        pm = (pcol == iota_p).astype(_F32)
        oh = (ncol == iota_n).astype(_F32)
        xg = _dot(oh, x2)
        tg = _dot(oh, t2)
        rg = _dot(oh, r2)
        xs = [jnp.sum(pm * xg[:, m * 16:(m + 1) * 16], axis=1) for m in range(3)]
        sq = None
        for c in range(3):
            pos = jnp.sum(pm * tg[:, c * 16:(c + 1) * 16], axis=1)
            for m in range(3):
                r_cm = jnp.sum(pm * rg[:, ((3 * c + m) * 16):((3 * c + m + 1) * 16)], axis=1)
                pos = pos + r_cm * xs[m]
            hv_c = jnp.sum(pm * hv[c, :].reshape(1, _P), axis=1)
            e = pos - hv_c
            sq = e * e if sq is None else sq + e * e
        return jnp.sum(sq) / (64.0 * 3.0)

    moving_loss = _handle_loss(mi_ref)
    static_loss = _handle_loss(si_ref)

    oi = _fiota((1, 128), 1)
    out_ref[...] = (jnp.where(oi == 0.0, arap, 0.0)
                    + jnp.where(oi == 1.0, moving_loss, 0.0)
                    + jnp.where(oi == 2.0, static_loss, 0.0))


def kernel(patch_verts, faces, rotations, translations, moving_idx, static_idx,
           handle_value, alternation):
    del alternation
    x2 = jnp.transpose(patch_verts, (1, 2, 0)).reshape(_N, 3 * _P)
    r2 = jnp.transpose(rotations, (1, 2, 3, 0)).reshape(_N, 9 * _P)
    t2 = jnp.transpose(translations, (1, 2, 0)).reshape(_N, 3 * _P)
    f = faces.T.astype(_F32)
    mi = moving_idx.T.astype(_F32)
    si = static_idx.T.astype(_F32)
    hv = handle_value.T
    out = pl.pallas_call(
        _body,
        out_shape=jax.ShapeDtypeStruct((1, 128), _F32),
        scratch_shapes=[
            pltpu.VMEM((_M // _FC, _FC), _F32),
            pltpu.VMEM((_M // _FC, _FC), _F32),
            pltpu.VMEM((_M // _FC, _FC), _F32),
            pltpu.VMEM((_E // _DC, _DC), _F32),
            pltpu.VMEM((_E // _DC, _DC), _F32),
            pltpu.VMEM((_E, _P), _F32),
            pltpu.VMEM((_E, _P), _F32),
        ],
    )(x2, r2, t2, f, mi, si, hv)
    return out[0, :3]


# speedup vs baseline: 10.4864x; 1.1372x over previous
import functools

import jax
import jax.numpy as jnp
from jax import lax
from jax.experimental import pallas as pl
from jax.experimental.pallas import tpu as pltpu
from jax.experimental.pallas import tpu_sc as plsc

_P, _N, _M = 16, 1024, 2048
_E = 3 * _M
_FC = 512
_EB = 256
_CH = 96

_F32 = jnp.float32
_DEF = jax.lax.Precision.DEFAULT


def _fiota(shape, dim):
    return jax.lax.broadcasted_iota(jnp.int32, shape, dim).astype(_F32)


def _dot(a, b, prec=_DEF):
    return jnp.dot(a, b, preferred_element_type=_F32, precision=prec)


def _dot_nt(a, b, prec=_DEF):
    return jax.lax.dot_general(a, b, (((1,), (1,)), ((), ())),
                               preferred_element_type=_F32, precision=prec)


def _split(v):
    hi = v.astype(jnp.bfloat16).astype(_F32)
    return hi, v - hi


def _dot2(onehot, hi, lo):
    return _dot(onehot, hi) + _dot(onehot, lo)


def _to_col(row):
    k = row.shape[1]
    ident = (_fiota((k, 1), 0) == _fiota((1, k), 1)).astype(_F32)
    return _dot_nt(ident, row)


def _ld(ref, e, ch):
    return ref[e, ch // 8, pl.ds((ch % 8) * 16, 16)]


def _sc_d_body(tab_hbm, ii_hbm, jj_hbm, d_hbm,
               ii_v, jj_v, jrow_v, irow_v, out_v, sem):
    cid = lax.axis_index("c")
    sid = lax.axis_index("s")
    info = plsc.get_sparse_core_info()
    nw = info.num_cores * info.num_subcores
    wid = sid * info.num_cores + cid
    per_w = _E // nw
    nch = per_w // _CH

    def chunk(c, carry):
        base = wid * per_w + c * _CH
        pltpu.sync_copy(ii_hbm.at[pl.ds(base, _CH)], ii_v)
        pltpu.sync_copy(jj_hbm.at[pl.ds(base, _CH)], jj_v)
        cpj = pltpu.make_async_copy(tab_hbm.at[jj_v], jrow_v, sem)
        cpi = pltpu.make_async_copy(tab_hbm.at[ii_v], irow_v, sem)
        cpj.start()
        cpi.start()
        cpj.wait()
        cpi.wait()

        def edge(e, carry2):
            xs = [_ld(jrow_v, e, m) for m in range(3)]
            acc = None
            for cc in range(3):
                dc = _ld(jrow_v, e, 3 + cc)
                for m in range(3):
                    ch = 6 + 3 * cc + m
                    dr = _ld(jrow_v, e, ch) - _ld(irow_v, e, ch)
                    dc = dc + dr * xs[m]
                acc = dc * dc if acc is None else acc + dc * dc
            out_v[e] = acc
            return carry2

        lax.fori_loop(0, _CH, edge, 0)
        pltpu.sync_copy(out_v, d_hbm.at[pl.ds(base, _CH)])
        return carry

    lax.fori_loop(0, nch, chunk, 0)


def _sc_d(tab, ii, jj):
    mesh = plsc.VectorSubcoreMesh(core_axis_name="c", subcore_axis_name="s")
    k = functools.partial(
        pl.kernel, mesh=mesh,
        out_type=jax.ShapeDtypeStruct((_E, _P), _F32),
        scratch_types=[
            pltpu.VMEM((_CH,), jnp.int32),
            pltpu.VMEM((_CH,), jnp.int32),
            pltpu.VMEM((_CH, 2, 128), _F32),
            pltpu.VMEM((_CH, 2, 128), _F32),
            pltpu.VMEM((_CH, _P), _F32),
            pltpu.SemaphoreType.DMA,
        ],
    )(_sc_d_body)
    return k(tab, ii, jj)


def _body(x2_ref, r2_ref, t2_ref, f_ref, mi_ref, si_ref, hv_ref, d_in_ref,
          out_ref, f0_ref, f1_ref, f2_ref, cot_ref):
    x2 = x2_ref[...]
    r2 = r2_ref[...]
    t2 = t2_ref[...]
    f = f_ref[...]
    hv = hv_ref[...]
    iota_n = _fiota((1, _N), 1)

    tr_blocks = []
    for c in range(3):
        acc = t2[:, c * 16:(c + 1) * 16]
        for m in range(3):
            acc = acc + r2[:, ((3 * c + m) * 16):((3 * c + m + 1) * 16)] * x2[:, m * 16:(m + 1) * 16]
        tr_blocks.append(acc)
    tr = jnp.concatenate(tr_blocks, axis=1)

    f0 = f[0:1, :]
    f1 = f[1:2, :]
    f2 = f[2:3, :]
    i_row = jnp.concatenate([f0, f1, f0], axis=1)
    j_row = jnp.concatenate([f1, f2, f2], axis=1)
    key_row = i_row * float(_N) + j_row
    for r in range(_M // _FC):
        f0_ref[r:r + 1, :] = f0[:, r * _FC:(r + 1) * _FC]
        f1_ref[r:r + 1, :] = f1[:, r * _FC:(r + 1) * _FC]
        f2_ref[r:r + 1, :] = f2[:, r * _FC:(r + 1) * _FC]

    def _cot_step(c, carry):
        v = []
        for fr in (f0_ref, f1_ref, f2_ref):
            idx = _to_col(fr[pl.ds(c, 1), :])
            oh = (idx == iota_n).astype(_F32)
            v.append(_dot(oh, x2))

        def _sidelen(a, b):
            s = None
            for cc in range(3):
                dd = a[:, cc * 16:(cc + 1) * 16] - b[:, cc * 16:(cc + 1) * 16]
                s = dd * dd if s is None else s + dd * dd
            return jnp.sqrt(s)

        la = _sidelen(v[1], v[2])
        lb = _sidelen(v[0], v[2])
        lc = _sidelen(v[0], v[1])
        s = 0.5 * (la + lb + lc)
        area = jnp.sqrt(jnp.maximum(s * (s - la) * (s - lb) * (s - lc), 1e-12))
        a2, b2, c2 = la * la, lb * lb, lc * lc
        cot_ref[pl.ds(c * _FC, _FC), :] = (b2 + c2 - a2) / area * 0.25
        cot_ref[pl.ds(_M + c * _FC, _FC), :] = (a2 + c2 - b2) / area * 0.25
        cot_ref[pl.ds(2 * _M + c * _FC, _FC), :] = (a2 + b2 - c2) / area * 0.25
        return carry

    jax.lax.fori_loop(0, _M // _FC, _cot_step, 0)

    d_hi, d_lo = _split(d_in_ref[...])
    iota_e = _fiota((1, _E), 1)
    ones_e = jnp.full((1, _E), 1.0, dtype=_F32)

    def _arap_step(b, acc):
        erow = b.astype(_F32) * float(_EB) + _fiota((_EB, 1), 0)
        sel = (iota_e == erow).astype(_F32)
        ib = _dot_nt(sel, i_row)
        jb = _dot_nt(sel, j_row)
        kb = ib * float(_N) + jb
        rkb = jb * float(_N) + ib
        eqf = (kb == key_row)
        eqr = (rkb == key_row)
        later = jnp.logical_and(eqf, iota_e > erow).astype(_F32)
        cnt_after = _dot_nt(later, ones_e)
        mask_last = (cnt_after == 0.0).astype(_F32)
        eqsum = eqf.astype(_F32) + eqr.astype(_F32)
        dsum = _dot2(eqsum, d_hi, d_lo)
        cot_b = cot_ref[pl.ds(b * _EB, _EB), :]
        contrib = jnp.sum(mask_last * cot_b * dsum, axis=0)
        return acc + contrib.reshape(1, _P)

    arap_acc = jax.lax.fori_loop(0, _E // _EB, _arap_step,
                                 jnp.zeros((1, _P), dtype=_F32))
    arap = jnp.sum(arap_acc) / float(_P)

    iota_p = _fiota((1, _P), 1)

    def _handle_loss(hidx_ref):
        pcol = _to_col(hidx_ref[0:1, :])
        ncol = _to_col(hidx_ref[1:2, :])
        pm = (pcol == iota_p).astype(_F32)
        oh = (ncol == iota_n).astype(_F32)
        trh, trl = _split(tr)
        tg = _dot2(oh, trh, trl)
        sq = None
        for c in range(3):
            pos = jnp.sum(pm * tg[:, c * 16:(c + 1) * 16], axis=1)
            hv_c = jnp.sum(pm * hv[c, :].reshape(1, _P), axis=1)
            e = pos - hv_c
            sq = e * e if sq is None else sq + e * e
        return jnp.sum(sq) / (64.0 * 3.0)

    moving_loss = _handle_loss(mi_ref)
    static_loss = _handle_loss(si_ref)

    oi = _fiota((1, 128), 1)
    out_ref[...] = (jnp.where(oi == 0.0, arap, 0.0)
                    + jnp.where(oi == 1.0, moving_loss, 0.0)
                    + jnp.where(oi == 2.0, static_loss, 0.0))


def kernel(patch_verts, faces, rotations, translations, moving_idx, static_idx,
           handle_value, alternation):
    del alternation
    tab = jnp.concatenate(
        [jnp.transpose(patch_verts, (1, 2, 0)),
         jnp.transpose(translations, (1, 2, 0)),
         jnp.transpose(rotations, (1, 2, 3, 0)).reshape(_N, 9, _P),
         jnp.zeros((_N, 1, _P), _F32)], axis=1).reshape(_N, 2, 128)
    ii = jnp.concatenate([faces[:, 0], faces[:, 1], faces[:, 0]], axis=0)
    jj = jnp.concatenate([faces[:, 1], faces[:, 2], faces[:, 2]], axis=0)
    d = _sc_d(tab, ii, jj)

    x2 = jnp.transpose(patch_verts, (1, 2, 0)).reshape(_N, 3 * _P)
    r2 = jnp.transpose(rotations, (1, 2, 3, 0)).reshape(_N, 9 * _P)
    t2 = jnp.transpose(translations, (1, 2, 0)).reshape(_N, 3 * _P)
    f = faces.T.astype(_F32)
    mi = moving_idx.T.astype(_F32)
    si = static_idx.T.astype(_F32)
    hv = handle_value.T
    out = pl.pallas_call(
        _body,
        out_shape=jax.ShapeDtypeStruct((1, 128), _F32),
        scratch_shapes=[
            pltpu.VMEM((_M // _FC, _FC), _F32),
            pltpu.VMEM((_M // _FC, _FC), _F32),
            pltpu.VMEM((_M // _FC, _FC), _F32),
            pltpu.VMEM((_E, _P), _F32),
        ],
    )(x2, r2, t2, f, mi, si, hv, d)
    return out[0, :3]


# speedup vs baseline: 10.5043x; 1.0017x over previous
import functools

import jax
import jax.numpy as jnp
from jax import lax
from jax.experimental import pallas as pl
from jax.experimental.pallas import tpu as pltpu
from jax.experimental.pallas import tpu_sc as plsc

_P, _N, _M = 16, 1024, 2048
_E = 3 * _M
_FC = 512
_EB = 256
_CH = 96

_F32 = jnp.float32
_DEF = jax.lax.Precision.DEFAULT


def _fiota(shape, dim):
    return jax.lax.broadcasted_iota(jnp.int32, shape, dim).astype(_F32)


def _dot(a, b, prec=_DEF):
    return jnp.dot(a, b, preferred_element_type=_F32, precision=prec)


def _dot_nt(a, b, prec=_DEF):
    return jax.lax.dot_general(a, b, (((1,), (1,)), ((), ())),
                               preferred_element_type=_F32, precision=prec)


def _split(v):
    hi = v.astype(jnp.bfloat16).astype(_F32)
    return hi, v - hi


def _dot2(onehot, hi, lo):
    return _dot(onehot, hi) + _dot(onehot, lo)


def _to_col(row):
    k = row.shape[1]
    ident = (_fiota((k, 1), 0) == _fiota((1, k), 1)).astype(_F32)
    return _dot_nt(ident, row)


def _ld(ref, e, ch):
    return ref[e, ch // 8, pl.ds((ch % 8) * 16, 16)]


def _sc_d_body(tab_hbm, ii_hbm, jj_hbm, d_hbm,
               ii_v, jj_v, jrow_v, irow_v, out_v, sem):
    cid = lax.axis_index("c")
    sid = lax.axis_index("s")
    info = plsc.get_sparse_core_info()
    nw = info.num_cores * info.num_subcores
    wid = sid * info.num_cores + cid
    per_w = _E // nw
    nch = per_w // _CH

    def chunk(c, carry):
        base = wid * per_w + c * _CH
        pltpu.sync_copy(ii_hbm.at[pl.ds(base, _CH)], ii_v)
        pltpu.sync_copy(jj_hbm.at[pl.ds(base, _CH)], jj_v)
        cpj = pltpu.make_async_copy(tab_hbm.at[jj_v], jrow_v, sem)
        cpi = pltpu.make_async_copy(tab_hbm.at[ii_v], irow_v, sem)
        cpj.start()
        cpi.start()
        cpj.wait()
        cpi.wait()

        def edge(e, carry2):
            xs = [_ld(jrow_v, e, m) for m in range(3)]
            acc = None
            for cc in range(3):
                dc = _ld(jrow_v, e, 3 + cc)
                for m in range(3):
                    ch = 6 + 3 * cc + m
                    dr = _ld(jrow_v, e, ch) - _ld(irow_v, e, ch)
                    dc = dc + dr * xs[m]
                acc = dc * dc if acc is None else acc + dc * dc
            out_v[e] = acc
            return carry2

        lax.fori_loop(0, _CH, edge, 0)
        pltpu.sync_copy(out_v, d_hbm.at[pl.ds(base, _CH)])
        return carry

    lax.fori_loop(0, nch, chunk, 0)


def _sc_d(tab, ii, jj):
    mesh = plsc.VectorSubcoreMesh(core_axis_name="c", subcore_axis_name="s")
    k = functools.partial(
        pl.kernel, mesh=mesh,
        out_type=jax.ShapeDtypeStruct((_E, _P), _F32),
        scratch_types=[
            pltpu.VMEM((_CH,), jnp.int32),
            pltpu.VMEM((_CH,), jnp.int32),
            pltpu.VMEM((_CH, 2, 128), _F32),
            pltpu.VMEM((_CH, 2, 128), _F32),
            pltpu.VMEM((_CH, _P), _F32),
            pltpu.SemaphoreType.DMA,
        ],
    )(_sc_d_body)
    return k(tab, ii, jj)


def _body(x2_ref, r2_ref, t2_ref, f_ref, mi_ref, si_ref, hv_ref, d_in_ref,
          out_ref, f0_ref, f1_ref, f2_ref, cot_ref):
    x2 = x2_ref[...]
    r2 = r2_ref[...]
    t2 = t2_ref[...]
    f = f_ref[...]
    hv = hv_ref[...]
    iota_n = _fiota((1, _N), 1)

    tr_blocks = []
    for c in range(3):
        acc = t2[:, c * 16:(c + 1) * 16]
        for m in range(3):
            acc = acc + r2[:, ((3 * c + m) * 16):((3 * c + m + 1) * 16)] * x2[:, m * 16:(m + 1) * 16]
        tr_blocks.append(acc)
    tr = jnp.concatenate(tr_blocks, axis=1)

    f0 = f[0:1, :]
    f1 = f[1:2, :]
    f2 = f[2:3, :]
    i_row = jnp.concatenate([f0, f1, f0], axis=1)
    j_row = jnp.concatenate([f1, f2, f2], axis=1)
    key_row = i_row * float(_N) + j_row
    for r in range(_M // _FC):
        f0_ref[r:r + 1, :] = f0[:, r * _FC:(r + 1) * _FC]
        f1_ref[r:r + 1, :] = f1[:, r * _FC:(r + 1) * _FC]
        f2_ref[r:r + 1, :] = f2[:, r * _FC:(r + 1) * _FC]

    def _cot_step(c, carry):
        v = []
        for fr in (f0_ref, f1_ref, f2_ref):
            idx = _to_col(fr[pl.ds(c, 1), :])
            oh = (idx == iota_n).astype(_F32)
            v.append(_dot(oh, x2))

        def _sidelen(a, b):
            s = None
            for cc in range(3):
                dd = a[:, cc * 16:(cc + 1) * 16] - b[:, cc * 16:(cc + 1) * 16]
                s = dd * dd if s is None else s + dd * dd
            return jnp.sqrt(s)

        la = _sidelen(v[1], v[2])
        lb = _sidelen(v[0], v[2])
        lc = _sidelen(v[0], v[1])
        s = 0.5 * (la + lb + lc)
        area = jnp.sqrt(jnp.maximum(s * (s - la) * (s - lb) * (s - lc), 1e-12))
        a2, b2, c2 = la * la, lb * lb, lc * lc
        cot_ref[pl.ds(c * _FC, _FC), :] = (b2 + c2 - a2) / area * 0.25
        cot_ref[pl.ds(_M + c * _FC, _FC), :] = (a2 + c2 - b2) / area * 0.25
        cot_ref[pl.ds(2 * _M + c * _FC, _FC), :] = (a2 + b2 - c2) / area * 0.25
        return carry

    jax.lax.fori_loop(0, _M // _FC, _cot_step, 0)

    d_hi, d_lo = _split(d_in_ref[...])
    iota_e = _fiota((1, _E), 1)
    ones_e = jnp.full((1, _E), 1.0, dtype=_F32)
    u_row = (jnp.minimum(i_row, j_row) * float(_N)
             + jnp.maximum(i_row, j_row))

    def _arap_step(b, acc):
        erow = b.astype(_F32) * float(_EB) + _fiota((_EB, 1), 0)
        sel = (iota_e == erow).astype(_F32)
        ib = _dot_nt(sel, i_row)
        jb = _dot_nt(sel, j_row)
        kb = ib * float(_N) + jb
        ub = (jnp.minimum(ib, jb) * float(_N) + jnp.maximum(ib, jb))
        eqf = (kb == key_row)
        equ = (ub == u_row).astype(_F32)
        later = jnp.logical_and(eqf, iota_e > erow).astype(_F32)
        cnt_after = _dot_nt(later, ones_e)
        mask_last = (cnt_after == 0.0).astype(_F32)
        dsum = _dot2(equ, d_hi, d_lo)
        selfb = 1.0 + (ib == jb).astype(_F32)
        cot_b = cot_ref[pl.ds(b * _EB, _EB), :]
        contrib = jnp.sum(mask_last * selfb * cot_b * dsum, axis=0)
        return acc + contrib.reshape(1, _P)

    arap_acc = jax.lax.fori_loop(0, _E // _EB, _arap_step,
                                 jnp.zeros((1, _P), dtype=_F32))
    arap = jnp.sum(arap_acc) / float(_P)

    iota_p = _fiota((1, _P), 1)

    def _handle_loss(hidx_ref):
        pcol = _to_col(hidx_ref[0:1, :])
        ncol = _to_col(hidx_ref[1:2, :])
        pm = (pcol == iota_p).astype(_F32)
        oh = (ncol == iota_n).astype(_F32)
        trh, trl = _split(tr)
        tg = _dot2(oh, trh, trl)
        sq = None
        for c in range(3):
            pos = jnp.sum(pm * tg[:, c * 16:(c + 1) * 16], axis=1)
            hv_c = jnp.sum(pm * hv[c, :].reshape(1, _P), axis=1)
            e = pos - hv_c
            sq = e * e if sq is None else sq + e * e
        return jnp.sum(sq) / (64.0 * 3.0)

    moving_loss = _handle_loss(mi_ref)
    static_loss = _handle_loss(si_ref)

    oi = _fiota((1, 128), 1)
    out_ref[...] = (jnp.where(oi == 0.0, arap, 0.0)
                    + jnp.where(oi == 1.0, moving_loss, 0.0)
                    + jnp.where(oi == 2.0, static_loss, 0.0))


def kernel(patch_verts, faces, rotations, translations, moving_idx, static_idx,
           handle_value, alternation):
    del alternation
    tab = jnp.concatenate(
        [jnp.transpose(patch_verts, (1, 2, 0)),
         jnp.transpose(translations, (1, 2, 0)),
         jnp.transpose(rotations, (1, 2, 3, 0)).reshape(_N, 9, _P),
         jnp.zeros((_N, 1, _P), _F32)], axis=1).reshape(_N, 2, 128)
    ii = jnp.concatenate([faces[:, 0], faces[:, 1], faces[:, 0]], axis=0)
    jj = jnp.concatenate([faces[:, 1], faces[:, 2], faces[:, 2]], axis=0)
    d = _sc_d(tab, ii, jj)

    x2 = jnp.transpose(patch_verts, (1, 2, 0)).reshape(_N, 3 * _P)
    r2 = jnp.transpose(rotations, (1, 2, 3, 0)).reshape(_N, 9 * _P)
    t2 = jnp.transpose(translations, (1, 2, 0)).reshape(_N, 3 * _P)
    f = faces.T.astype(_F32)
    mi = moving_idx.T.astype(_F32)
    si = static_idx.T.astype(_F32)
    hv = handle_value.T
    out = pl.pallas_call(
        _body,
        out_shape=jax.ShapeDtypeStruct((1, 128), _F32),
        scratch_shapes=[
            pltpu.VMEM((_M // _FC, _FC), _F32),
            pltpu.VMEM((_M // _FC, _FC), _F32),
            pltpu.VMEM((_M // _FC, _FC), _F32),
            pltpu.VMEM((_E, _P), _F32),
        ],
    )(x2, r2, t2, f, mi, si, hv, d)
    return out[0, :3]
